# Initial kernel scaffold; baseline (speedup 1.0000x reference)
import jax
import jax.numpy as jnp
from jax.experimental import pallas as pl


def kernel(atom_features, edge_index, pair_indices, pair_features, W_emb, b_emb, conv_W, conv_b, bn_gamma, bn_beta, mlp_W1, mlp_b1, mlp_W2, mlp_b2, mlp_W3, mlp_b3):
    raise NotImplementedError("write your pallas kernel here")


# speedup vs baseline: 4.0817x; 4.0817x over previous
import functools

import jax
import jax.numpy as jnp
from jax import lax
from jax.experimental import pallas as pl
from jax.experimental.pallas import tpu as pltpu
from jax.experimental.pallas import tpu_sc as plsc

N = 10000
E = 320000
P = 50000
H = 128
D_PAIR = 16

NC = 2
NS = 16
NW = NC * NS
CH = 128
WH = 16

NPAD = 10240
ROWS_PER_TILE = NPAD // NS
TRASH = 10112
ECH = 80
EPAD = NW * ECH * CH
PCH = 32
PHALF = (NW * PCH * CH) // 2
GROWS = 2 * PHALF
MROWS = 53248

_mesh = plsc.VectorSubcoreMesh(core_axis_name="c", subcore_axis_name="s")


@functools.partial(
    pl.kernel,
    out_type=jax.ShapeDtypeStruct((NC * NPAD, WH), jnp.float32),
    mesh=_mesh,
    scratch_types=[
        pltpu.VMEM((ECH, CH), jnp.int32),
        pltpu.VMEM((CH, WH), jnp.float32),
        pltpu.VMEM((CH, WH), jnp.float32),
        pltpu.VMEM_SHARED((NPAD, WH), jnp.float32),
    ],
)
def _sc_hist(dst_hbm, ones_hbm, z_hbm, out_hbm, dst_v, ones_v, z_v, acc):
    cid = lax.axis_index("c")
    sid = lax.axis_index("s")
    wid = cid * NS + sid
    pltpu.sync_copy(dst_hbm.at[pl.ds(wid * ECH, ECH)], dst_v)
    pltpu.sync_copy(ones_hbm, ones_v)
    pltpu.sync_copy(z_hbm, z_v)
    for z in range(ROWS_PER_TILE // CH):
        pltpu.sync_copy(z_v, acc.at[pl.ds(sid * ROWS_PER_TILE + z * CH, CH)])
    plsc.subcore_barrier()

    def body(j, carry):
        pltpu.sync_copy(ones_v, acc.at[dst_v.at[j]], add=True)
        return carry

    lax.fori_loop(0, ECH, body, 0)
    plsc.subcore_barrier()
    pltpu.sync_copy(
        acc.at[pl.ds(sid * ROWS_PER_TILE, ROWS_PER_TILE)],
        out_hbm.at[pl.ds(cid * NPAD + sid * ROWS_PER_TILE, ROWS_PER_TILE)],
    )


QCH = 16


@functools.partial(
    pl.kernel,
    out_type=jax.ShapeDtypeStruct((NC * NPAD, H), jnp.float32),
    mesh=_mesh,
    scratch_types=[
        pltpu.VMEM((QCH, CH), jnp.int32),
        pltpu.VMEM((QCH, CH), jnp.int32),
        pltpu.VMEM((CH, H), jnp.float32),
        pltpu.VMEM((CH, H), jnp.float32),
        pltpu.VMEM_SHARED((NPAD, H), jnp.float32),
        pltpu.SemaphoreType.DMA,
        pltpu.SemaphoreType.DMA,
    ],
)
def _sc_scatter(g_hbm, src_hbm, dst_hbm, z_hbm, out_hbm,
                src_v, dst_v, buf0, buf1, acc, sem0, sem1):
    cid = lax.axis_index("c")
    sid = lax.axis_index("s")
    wid = cid * NS + sid
    pltpu.sync_copy(z_hbm, buf0)
    for z in range(ROWS_PER_TILE // CH):
        pltpu.sync_copy(buf0, acc.at[pl.ds(sid * ROWS_PER_TILE + z * CH, CH)])
    plsc.subcore_barrier()

    def quarter(q, carry):
        qbase = wid * ECH + q * QCH
        pltpu.sync_copy(src_hbm.at[pl.ds(qbase, QCH)], src_v)
        pltpu.sync_copy(dst_hbm.at[pl.ds(qbase, QCH)], dst_v)

        def body(j, c):
            pltpu.async_copy(g_hbm.at[src_v.at[j]], buf1, sem1).wait()
            pltpu.sync_copy(buf1, acc.at[dst_v.at[j]], add=True)
            return c

        return lax.fori_loop(0, QCH, body, carry)

    lax.fori_loop(0, ECH // QCH, quarter, 0)
    plsc.subcore_barrier()
    pltpu.sync_copy(
        acc.at[pl.ds(sid * ROWS_PER_TILE, ROWS_PER_TILE)],
        out_hbm.at[pl.ds(cid * NPAD + sid * ROWS_PER_TILE, ROWS_PER_TILE)],
    )


@functools.partial(
    pl.kernel,
    out_type=jax.ShapeDtypeStruct((GROWS, H), jnp.float32),
    mesh=_mesh,
    scratch_types=[
        pltpu.VMEM((PCH, CH), jnp.int32),
        pltpu.VMEM((CH, H), jnp.float32),
        pltpu.SemaphoreType.DMA,
    ],
)
def _sc_gather(x_hbm, idx_hbm, out_hbm, idx_v, buf0, sem0):
    cid = lax.axis_index("c")
    sid = lax.axis_index("s")
    wid = cid * NS + sid
    base = wid * PCH
    pltpu.sync_copy(idx_hbm.at[pl.ds(base, PCH)], idx_v)

    def body(j, carry):
        pltpu.async_copy(x_hbm.at[idx_v.at[j]], buf0, sem0).wait()
        pltpu.sync_copy(buf0, out_hbm.at[pl.ds((base + j) * CH, CH)])
        return carry

    lax.fori_loop(0, PCH, body, 0)


def _tc_emb_body(atom_ref, wemb_ref, bemb_ref, hist_ref, w0_ref, g0_ref, dis_ref):
    x = jnp.dot(atom_ref[...], wemb_ref[...],
                preferred_element_type=jnp.float32) + bemb_ref[...]
    hs = hist_ref[:NPAD, 0:1] + hist_ref[NPAD:, 0:1]
    dis = lax.rsqrt(1.0 + hs)
    dis_ref[...] = dis
    g0_ref[...] = dis * jnp.dot(x, w0_ref[...], preferred_element_type=jnp.float32)


def _tc_emb(atom_p, w_emb, b_emb, hist, w0):
    return pl.pallas_call(
        _tc_emb_body,
        out_shape=(
            jax.ShapeDtypeStruct((NPAD, H), jnp.float32),
            jax.ShapeDtypeStruct((NPAD, 1), jnp.float32),
        ),
    )(atom_p, w_emb, b_emb, hist, w0)


def _tc_layer_body(parts_ref, g_ref, dis_ref, b_ref, gam_ref, bet_ref, w_ref,
                   out_ref, *, last):
    dis = dis_ref[...]
    pre = dis * (parts_ref[:NPAD] + parts_ref[NPAD:] + g_ref[...]) + b_ref[...]
    real = pre[:N]
    mu = jnp.mean(real, axis=0, keepdims=True)
    var = jnp.mean((real - mu) ** 2, axis=0, keepdims=True)
    xn = (pre - mu) * lax.rsqrt(var + 1e-5) * gam_ref[...] + bet_ref[...]
    xn = jnp.maximum(xn, 0.0)
    if last:
        out_ref[...] = xn
    else:
        out_ref[...] = dis * jnp.dot(xn, w_ref[...],
                                     preferred_element_type=jnp.float32)


def _tc_layer(parts, g, dis, b, gam, bet, w, last):
    return pl.pallas_call(
        functools.partial(_tc_layer_body, last=last),
        out_shape=jax.ShapeDtypeStruct((NPAD, H), jnp.float32),
    )(parts, g, dis, b, gam, bet, w)


def _tc_mlp_body(a0_ref, a1_ref, pf_ref, w1a_ref, w1b_ref, w1c_ref, b1_ref,
                 w2_ref, b2_ref, w3_ref, b3_ref, out_ref):
    h1 = (jnp.dot(a0_ref[...], w1a_ref[...], preferred_element_type=jnp.float32)
          + jnp.dot(a1_ref[...], w1b_ref[...], preferred_element_type=jnp.float32)
          + jnp.dot(pf_ref[...], w1c_ref[...], preferred_element_type=jnp.float32)
          + b1_ref[...])
    h1 = jnp.maximum(h1, 0.0)
    h2 = jnp.maximum(jnp.dot(h1, w2_ref[...],
                             preferred_element_type=jnp.float32) + b2_ref[...], 0.0)
    out_ref[...] = jnp.dot(h2, w3_ref[...],
                           preferred_element_type=jnp.float32) + b3_ref[...]


def _tc_mlp(arows, pf_p, w1a, w1b, w1c, b1, w2, b2, w3, b3):
    blk = 4096
    nblk = MROWS // blk
    half = PHALF // blk
    wspec = lambda shape: pl.BlockSpec(shape, lambda i: (0, 0))
    return pl.pallas_call(
        _tc_mlp_body,
        grid=(nblk,),
        in_specs=[
            pl.BlockSpec((blk, H), lambda i: (i, 0)),
            pl.BlockSpec((blk, H), lambda i: (i + half, 0)),
            pl.BlockSpec((blk, D_PAIR), lambda i: (i, 0)),
            wspec((H, H)), wspec((H, H)), wspec((D_PAIR, H)), wspec((1, H)),
            wspec((H, H // 2)), wspec((1, H // 2)),
            wspec((H // 2, 1)), wspec((1, 1)),
        ],
        out_specs=pl.BlockSpec((blk, 1), lambda i: (i, 0)),
        out_shape=jax.ShapeDtypeStruct((MROWS, 1), jnp.float32),
    )(arows, arows, pf_p, w1a, w1b, w1c, b1, w2, b2, w3, b3)


def kernel(atom_features, edge_index, pair_indices, pair_features, W_emb, b_emb,
           conv_W, conv_b, bn_gamma, bn_beta, mlp_W1, mlp_b1, mlp_W2, mlp_b2,
           mlp_W3, mlp_b3):
    f32 = jnp.float32
    epad = jnp.full((EPAD - E,), TRASH, jnp.int32)
    srcp = jnp.concatenate([edge_index[0], epad]).reshape(NW * ECH, CH)
    dstp = jnp.concatenate([edge_index[1], epad]).reshape(NW * ECH, CH)

    ones16 = jnp.ones((CH, WH), f32)
    z16 = jnp.zeros((CH, WH), f32)
    z128 = jnp.zeros((CH, H), f32)

    hist = _sc_hist(dstp, ones16, z16)

    atom_p = jnp.pad(atom_features, ((0, NPAD - N), (0, 0)))
    g, dis = _tc_emb(atom_p, W_emb, b_emb.reshape(1, H), hist, conv_W[0])

    for i in range(3):
        parts = _sc_scatter(g, srcp, dstp, z128)
        last = i == 2
        g = _tc_layer(parts, g, dis, conv_b[i].reshape(1, H),
                      bn_gamma[i].reshape(1, H), bn_beta[i].reshape(1, H),
                      conv_W[i + 1] if not last else conv_W[0], last)

    ppad = jnp.zeros((PHALF - P,), jnp.int32)
    gidx = jnp.concatenate(
        [pair_indices[:, 0], ppad, pair_indices[:, 1], ppad]
    ).reshape(NW * PCH, CH)
    arows = _sc_gather(g, gidx)

    pf_p = jnp.pad(pair_features, ((0, MROWS - P), (0, 0)))
    out = _tc_mlp(arows, pf_p,
                  mlp_W1[:H], mlp_W1[H:2 * H], mlp_W1[2 * H:],
                  mlp_b1.reshape(1, H), mlp_W2, mlp_b2.reshape(1, H // 2),
                  mlp_W3, mlp_b3.reshape(1, 1))
    return out[:P]
